# baseline (speedup 1.0000x reference)
import functools

import jax
import jax.numpy as jnp
import numpy as np
from jax import lax
from jax.experimental import pallas as pl
from jax.experimental.pallas import tpu as pltpu
from jax.experimental.pallas import tpu_sc as plsc

D_MODEL = 2048
N_HEADS = 16
N_SELECTED = 16
INDEX_HEAD_DIM = 32
N_IDX = N_HEADS * INDEX_HEAD_DIM
B = 16
TQ = 8
TKV = 8192
D_HEAD = 128
ROWS = N_HEADS * TQ

KV_TILE = 512
NKV = TKV // KV_TILE

NROWS = B * ROWS
NW = 32
ROWS_PER_W = NROWS // NW
NBINS = 256
RBLK = 8
CAP = TKV + 16
NEG_INF = np.float32(-np.inf)


def _tc_body(qf_ref, wqt_ref, wkt_ref, k_ref, s_out_ref, qbd_ref):
    j = pl.program_id(1)
    scale = np.float32(1.0 / np.sqrt(INDEX_HEAD_DIM))

    @pl.when(j == 0)
    def _init():
        qi = jax.lax.dot_general(
            qf_ref[0], wqt_ref[...], (((1,), (0,)), ((), ())),
            preferred_element_type=jnp.float32)
        qi_t = jnp.concatenate([qi] * N_HEADS, axis=0)
        rblk = jax.lax.broadcasted_iota(jnp.int32, (ROWS, N_IDX), 0) // TQ
        cblk = jax.lax.broadcasted_iota(jnp.int32, (ROWS, N_IDX), 1) // INDEX_HEAD_DIM
        qbd_ref[...] = jnp.where(rblk == cblk, qi_t, jnp.float32(0))

    k = k_ref[0]
    kidx = jnp.zeros((KV_TILE, N_IDX), jnp.float32)
    for h2 in range(N_HEADS // 2):
        kpair = jnp.concatenate([k[2 * h2], k[2 * h2 + 1]], axis=1)
        kidx += jax.lax.dot_general(
            kpair, wkt_ref[h2 * 2 * D_HEAD:(h2 + 1) * 2 * D_HEAD, :],
            (((1,), (0,)), ((), ())),
            preferred_element_type=jnp.float32)

    s = jax.lax.dot_general(
        qbd_ref[...], kidx, (((1,), (1,)), ((), ())),
        preferred_element_type=jnp.float32)
    s_out_ref[0] = s * scale


NCH = 4
BCH = B // NCH


def _tc_scores_chunk(c, qf, wqt, wkt, key_states):
    return pl.pallas_call(
        _tc_body,
        grid=(BCH, NKV),
        in_specs=[
            pl.BlockSpec((1, TQ, D_MODEL), lambda b, j: (c * BCH + b, 0, 0)),
            pl.BlockSpec((D_MODEL, N_IDX), lambda b, j: (0, 0)),
            pl.BlockSpec((D_MODEL, N_IDX), lambda b, j: (0, 0)),
            pl.BlockSpec((1, N_HEADS, KV_TILE, D_HEAD),
                         lambda b, j: (c * BCH + b, 0, j, 0)),
        ],
        out_specs=pl.BlockSpec((1, ROWS, KV_TILE), lambda b, j: (b, 0, j)),
        out_shape=jax.ShapeDtypeStruct((BCH, ROWS, TKV), jnp.float32),
        scratch_shapes=[pltpu.VMEM((ROWS, N_IDX), jnp.float32)],
        compiler_params=pltpu.CompilerParams(
            dimension_semantics=("arbitrary", "arbitrary"),
        ),
    )(qf, wqt, wkt, key_states)


NROWS_CH = BCH * ROWS
RPW_CH = NROWS_CH // NW


@functools.partial(
    pl.kernel,
    mesh=plsc.VectorSubcoreMesh(core_axis_name="c", subcore_axis_name="s"),
    out_type=[
        jax.ShapeDtypeStruct((NROWS_CH, N_SELECTED), jnp.int32),
        jax.ShapeDtypeStruct((NROWS_CH, N_SELECTED), jnp.float32),
    ],
    scratch_types=[
        pltpu.VMEM((RBLK, TKV), jnp.float32),
        pltpu.VMEM((CAP,), jnp.float32),
        pltpu.VMEM((CAP,), jnp.int32),
        pltpu.VMEM((16 * NBINS,), jnp.int32),
        pltpu.VMEM((NBINS,), jnp.int32),
        pltpu.VMEM((N_SELECTED,), jnp.float32),
        pltpu.VMEM((N_SELECTED,), jnp.int32),
    ],
    compiler_params=pltpu.CompilerParams(needs_layout_passes=False),
)
def _sc_topk(scores_hbm, out_i_hbm, out_v_hbm,
             rows_v, cand_v, cand_i, hist_v, tot_v, tv_v, ti_v):
    wid = lax.axis_index("s") * 2 + lax.axis_index("c")
    lanes = lax.iota(jnp.int32, 16)
    zeros16 = jnp.zeros((16,), jnp.int32)
    ones16 = jnp.ones((16,), jnp.int32)
    neginf16 = jnp.full((16,), NEG_INF, jnp.float32)
    big16 = jnp.full((16,), TKV, jnp.int32)

    def mono_bin(v):
        bits = lax.bitcast_convert_type(v, jnp.int32)
        key = jnp.where(v < 0.0, ~bits, bits ^ jnp.int32(-2**31))
        return lax.shift_right_logical(key, 24)

    def do_row(t, _):
        blk = t // RBLK
        r_in = t - blk * RBLK

        @pl.when(r_in == 0)
        def _fetch():
            pltpu.sync_copy(
                scores_hbm.at[pl.ds(wid * RPW_CH + blk * RBLK, RBLK)],
                rows_v)

        row = wid * RPW_CH + t

        def clr(g, _c):
            for u in range(4):
                hist_v[pl.ds((g * 4 + u) * 16, 16)] = zeros16
            return 0
        lax.fori_loop(0, NBINS // 4, clr, 0)

        def h1(g, _c):
            for u in range(4):
                v = rows_v[r_in, pl.ds((g * 4 + u) * 16, 16)]
                slot = lanes * NBINS + mono_bin(v)
                plsc.addupdate_scatter(hist_v, [slot], ones16)
            return 0
        lax.fori_loop(0, TKV // 64, h1, 0)

        def t1(g, _c):
            acc = zeros16
            for c in range(16):
                acc = acc + hist_v[pl.ds(c * NBINS + g * 16, 16)]
            tot_v[pl.ds(g * 16, 16)] = acc
            return 0
        lax.fori_loop(0, NBINS // 16, t1, 0)

        def sb(tg, carry):
            above, beta, found = carry
            g = 15 - tg
            v = tot_v[pl.ds(g * 16, 16)]
            rv = lax.rev(v, (0,))
            cs = plsc.cumsum(rv) + above
            ok = cs >= N_SELECTED
            ncross = plsc.all_reduce_population_count(ok)
            ffs = plsc.all_reduce_ffs(ok)
            beta_g = 15 - ffs + g * 16
            use = jnp.logical_and(jnp.logical_not(found), ncross > 0)
            beta = jnp.where(use, beta_g, beta)
            found = jnp.logical_or(found, ncross > 0)
            above = above + jnp.sum(v)
            return (above, beta, found)

        _, beta, _ = lax.fori_loop(
            0, NBINS // 16, sb,
            (jnp.int32(0), zeros16, jnp.zeros((16,), jnp.bool_)))

        key_min = lax.shift_left(beta, 24)
        tau_bits = jnp.where(beta >= 128, key_min ^ jnp.int32(-2**31),
                             ~key_min)
        tau = lax.bitcast_convert_type(tau_bits, jnp.float32)

        def c1(g4, off):
            for u in range(4):
                g = g4 * 4 + u
                v = rows_v[r_in, pl.ds(g * 16, 16)]
                m = v >= tau
                plsc.store_compressed(cand_v.at[pl.ds(off, 16)], v, mask=m)
                plsc.store_compressed(cand_i.at[pl.ds(off, 16)],
                                      g * 16 + lanes, mask=m)
                off = off + plsc.all_reduce_population_count(m)[0]
            return off
        cnt = lax.fori_loop(0, TKV // 64, c1, jnp.int32(0))

        cand_v[pl.ds(cnt, 16)] = neginf16
        cand_i[pl.ds(cnt, 16)] = big16
        ngroups = (cnt + 15) // 16

        def extract(r, carry):
            pm, pi = carry

            def scan_g(g, c2):
                bv, bi = c2
                v = cand_v[pl.ds(g * 16, 16)]
                i = cand_i[pl.ds(g * 16, 16)]
                elig = jnp.logical_or(v < pm,
                                      jnp.logical_and(v == pm, i > pi))
                v2 = jnp.where(elig, v, NEG_INF)
                better = jnp.logical_or(v2 > bv,
                                        jnp.logical_and(v2 == bv, i < bi))
                return (jnp.where(better, v2, bv), jnp.where(better, i, bi))

            bv, bi = lax.fori_loop(0, ngroups, scan_g, (neginf16, big16))
            m = jnp.max(bv)
            mi = jnp.min(jnp.where(bv == m, bi, TKV))
            tv_v[...] = jnp.where(lanes == r, m, tv_v[...])
            ti_v[...] = jnp.where(lanes == r, mi, ti_v[...])
            return (m, mi)

        lax.fori_loop(0, N_SELECTED, extract,
                      (jnp.float32(np.inf), jnp.int32(-1)))

        pltpu.sync_copy(ti_v, out_i_hbm.at[row])
        pltpu.sync_copy(tv_v, out_v_hbm.at[row])
        return 0

    lax.fori_loop(0, RPW_CH, do_row, 0)


@jax.jit
def kernel(query_states, key_states, W_q, W_k):
    query_flat = jnp.transpose(query_states, (0, 2, 1, 3)).reshape(B, TQ, D_MODEL)
    wqt, wkt = W_q.T, W_k.T
    ois, ovs = [], []
    for c in range(NCH):
        sc = _tc_scores_chunk(c, query_flat, wqt, wkt, key_states)
        oi, ov = _sc_topk(sc.reshape(NROWS_CH, TKV))
        ois.append(oi)
        ovs.append(ov)
    out_i = jnp.concatenate(ois, axis=0)
    out_v = jnp.concatenate(ovs, axis=0)
    top_indices = out_i.reshape(B, N_HEADS, TQ, N_SELECTED)
    top_scores = out_v.reshape(B, N_HEADS, TQ, N_SELECTED)
    return (top_indices, top_scores)


# speedup vs baseline: 1.5291x; 1.5291x over previous
import jax
import jax.numpy as jnp
import numpy as np
from jax.experimental import pallas as pl
from jax.experimental.pallas import tpu as pltpu

D_MODEL = 2048
N_HEADS = 16
N_SELECTED = 16
INDEX_HEAD_DIM = 32
N_IDX = N_HEADS * INDEX_HEAD_DIM
B = 16
TQ = 8
TKV = 8192
D_HEAD = 128
ROWS = N_HEADS * TQ

KV_TILE = 1024
NKV = TKV // KV_TILE


def _body(qf_ref, wqt_ref, wkt_ref, k_ref, out_i_ref, out_v_ref,
          qbd_ref, s_ref):
    j = pl.program_id(1)
    scale = np.float32(1.0 / np.sqrt(INDEX_HEAD_DIM))

    @pl.when(j == 0)
    def _init():
        qi = jax.lax.dot_general(
            qf_ref[0], wqt_ref[...], (((1,), (0,)), ((), ())),
            preferred_element_type=jnp.float32)
        qi_t = jnp.concatenate([qi] * N_HEADS, axis=0)
        rblk = jax.lax.broadcasted_iota(jnp.int32, (ROWS, N_IDX), 0) // TQ
        cblk = jax.lax.broadcasted_iota(jnp.int32, (ROWS, N_IDX), 1) // INDEX_HEAD_DIM
        qbd = jnp.where(rblk == cblk, qi_t, jnp.float32(0))
        qbd_ref[...] = qbd

    k = k_ref[0]
    kidx = jnp.zeros((KV_TILE, N_IDX), jnp.float32)
    for h2 in range(N_HEADS // 2):
        kpair = jnp.concatenate([k[2 * h2], k[2 * h2 + 1]], axis=1)
        kidx += jax.lax.dot_general(
            kpair, wkt_ref[h2 * 2 * D_HEAD:(h2 + 1) * 2 * D_HEAD, :],
            (((1,), (0,)), ((), ())),
            preferred_element_type=jnp.float32)

    s = jax.lax.dot_general(
        qbd_ref[...], kidx, (((1,), (1,)), ((), ())),
        preferred_element_type=jnp.float32)
    s_ref[:, pl.ds(j * KV_TILE, KV_TILE)] = s * scale

    @pl.when(j == NKV - 1)
    def _topk():
        idxs = jax.lax.broadcasted_iota(jnp.int32, (ROWS, TKV), 1)
        vals, sels = [], []
        for _ in range(N_SELECTED):
            work = s_ref[...]
            m = jnp.max(work, axis=1, keepdims=True)
            sel = jnp.min(jnp.where(work == m, idxs, TKV), axis=1,
                          keepdims=True)
            vals.append(m)
            sels.append(sel)
            s_ref[...] = jnp.where(idxs == sel, -jnp.inf, work)
        out_v_ref[0] = jnp.concatenate(vals, axis=1)
        out_i_ref[0] = jnp.concatenate(sels, axis=1)


@jax.jit
def kernel(query_states, key_states, W_q, W_k):
    query_flat = jnp.transpose(query_states, (0, 2, 1, 3)).reshape(B, TQ, D_MODEL)
    qf_b = query_flat
    wqt_b = W_q.T
    wkt_b = W_k.T

    out_i, out_v = pl.pallas_call(
        _body,
        grid=(B, NKV),
        in_specs=[
            pl.BlockSpec((1, TQ, D_MODEL), lambda b, j: (b, 0, 0)),
            pl.BlockSpec((D_MODEL, N_IDX), lambda b, j: (0, 0)),
            pl.BlockSpec((D_MODEL, N_IDX), lambda b, j: (0, 0)),
            pl.BlockSpec((1, N_HEADS, KV_TILE, D_HEAD), lambda b, j: (b, 0, j, 0)),
        ],
        out_specs=[
            pl.BlockSpec((1, ROWS, N_SELECTED), lambda b, j: (b, 0, 0)),
            pl.BlockSpec((1, ROWS, N_SELECTED), lambda b, j: (b, 0, 0)),
        ],
        out_shape=[
            jax.ShapeDtypeStruct((B, ROWS, N_SELECTED), jnp.int32),
            jax.ShapeDtypeStruct((B, ROWS, N_SELECTED), jnp.float32),
        ],
        scratch_shapes=[
            pltpu.VMEM((ROWS, N_IDX), jnp.float32),
            pltpu.VMEM((ROWS, TKV), jnp.float32),
        ],
        compiler_params=pltpu.CompilerParams(
            dimension_semantics=("arbitrary", "arbitrary"),
        ),
    )(qf_b, wqt_b, wkt_b, key_states)

    top_indices = out_i.reshape(B, N_HEADS, TQ, N_SELECTED)
    top_scores = out_v.reshape(B, N_HEADS, TQ, N_SELECTED)
    return (top_indices, top_scores)


# speedup vs baseline: 1.5991x; 1.0458x over previous
import jax
import jax.numpy as jnp
import numpy as np
from jax.experimental import pallas as pl
from jax.experimental.pallas import tpu as pltpu

D_MODEL = 2048
N_HEADS = 16
N_SELECTED = 16
INDEX_HEAD_DIM = 32
N_IDX = N_HEADS * INDEX_HEAD_DIM
B = 16
TQ = 8
TKV = 8192
D_HEAD = 128
ROWS = N_HEADS * TQ

KV_TILE = 2048
NKV = TKV // KV_TILE


def _body(qf_ref, wqt_ref, wkt_ref, k_ref, out_i_ref, out_v_ref,
          qbd_ref, s_ref):
    j = pl.program_id(1)
    scale = np.float32(1.0 / np.sqrt(INDEX_HEAD_DIM))

    @pl.when(j == 0)
    def _init():
        qi = jax.lax.dot_general(
            qf_ref[0], wqt_ref[...], (((1,), (0,)), ((), ())),
            preferred_element_type=jnp.float32)
        qi_t = jnp.concatenate([qi] * N_HEADS, axis=0)
        rblk = jax.lax.broadcasted_iota(jnp.int32, (ROWS, N_IDX), 0) // TQ
        cblk = jax.lax.broadcasted_iota(jnp.int32, (ROWS, N_IDX), 1) // INDEX_HEAD_DIM
        qbd = jnp.where(rblk == cblk, qi_t, jnp.float32(0))
        qbd_ref[...] = qbd

    k = k_ref[0]
    kidx = jnp.zeros((KV_TILE, N_IDX), jnp.float32)
    for h2 in range(N_HEADS // 2):
        kpair = jnp.concatenate([k[2 * h2], k[2 * h2 + 1]], axis=1)
        kidx += jax.lax.dot_general(
            kpair, wkt_ref[h2 * 2 * D_HEAD:(h2 + 1) * 2 * D_HEAD, :],
            (((1,), (0,)), ((), ())),
            preferred_element_type=jnp.float32)

    s = jax.lax.dot_general(
        qbd_ref[...], kidx, (((1,), (1,)), ((), ())),
        preferred_element_type=jnp.float32)
    s_ref[:, pl.ds(j * KV_TILE, KV_TILE)] = s * scale

    @pl.when(j == NKV - 1)
    def _topk():
        idxs = jax.lax.broadcasted_iota(jnp.int32, (ROWS, TKV), 1)
        vals, sels = [], []
        for _ in range(N_SELECTED):
            work = s_ref[...]
            m = jnp.max(work, axis=1, keepdims=True)
            sel = jnp.min(jnp.where(work == m, idxs, TKV), axis=1,
                          keepdims=True)
            vals.append(m)
            sels.append(sel)
            s_ref[...] = jnp.where(idxs == sel, -jnp.inf, work)
        out_v_ref[0] = jnp.concatenate(vals, axis=1)
        out_i_ref[0] = jnp.concatenate(sels, axis=1)


@jax.jit
def kernel(query_states, key_states, W_q, W_k):
    query_flat = jnp.transpose(query_states, (0, 2, 1, 3)).reshape(B, TQ, D_MODEL)
    qf_b = query_flat
    wqt_b = W_q.T
    wkt_b = W_k.T

    out_i, out_v = pl.pallas_call(
        _body,
        grid=(B, NKV),
        in_specs=[
            pl.BlockSpec((1, TQ, D_MODEL), lambda b, j: (b, 0, 0)),
            pl.BlockSpec((D_MODEL, N_IDX), lambda b, j: (0, 0)),
            pl.BlockSpec((D_MODEL, N_IDX), lambda b, j: (0, 0)),
            pl.BlockSpec((1, N_HEADS, KV_TILE, D_HEAD), lambda b, j: (b, 0, j, 0)),
        ],
        out_specs=[
            pl.BlockSpec((1, ROWS, N_SELECTED), lambda b, j: (b, 0, 0)),
            pl.BlockSpec((1, ROWS, N_SELECTED), lambda b, j: (b, 0, 0)),
        ],
        out_shape=[
            jax.ShapeDtypeStruct((B, ROWS, N_SELECTED), jnp.int32),
            jax.ShapeDtypeStruct((B, ROWS, N_SELECTED), jnp.float32),
        ],
        scratch_shapes=[
            pltpu.VMEM((ROWS, N_IDX), jnp.float32),
            pltpu.VMEM((ROWS, TKV), jnp.float32),
        ],
        compiler_params=pltpu.CompilerParams(
            dimension_semantics=("arbitrary", "arbitrary"),
        ),
    )(qf_b, wqt_b, wkt_b, key_states)

    top_indices = out_i.reshape(B, N_HEADS, TQ, N_SELECTED)
    top_scores = out_v.reshape(B, N_HEADS, TQ, N_SELECTED)
    return (top_indices, top_scores)
